# Initial kernel scaffold; baseline (speedup 1.0000x reference)
import jax
import jax.numpy as jnp
from jax.experimental import pallas as pl


def kernel(input_ids, base_weight, bit_proj_w):
    raise NotImplementedError("write your pallas kernel here")


# speedup vs baseline: 1.4633x; 1.4633x over previous
import functools

import jax
import jax.numpy as jnp
from jax import lax
from jax.experimental import pallas as pl
from jax.experimental.pallas import tpu as pltpu
from jax.experimental.pallas import tpu_sc as plsc

D = 1024
V = 256
NC, NS = 2, 16
NW = NC * NS
CHUNK = 32


def _table_body(base_ref, proj_ref, w_ref):
    r = lax.broadcasted_iota(jnp.int32, (V, 8), 0)
    j = lax.broadcasted_iota(jnp.int32, (V, 8), 1)
    bits = ((r >> (7 - j)) & 1).astype(jnp.float32)
    w_ref[...] = base_ref[...] + lax.dot_general(
        bits, proj_ref[...], (((1,), (1,)), ((), ())),
        preferred_element_type=jnp.float32)


def _build_table(base_weight, bit_proj_w):
    return pl.pallas_call(
        _table_body,
        out_shape=jax.ShapeDtypeStruct((V, D), jnp.float32),
    )(base_weight, bit_proj_w)


def _make_gather(total_ids):
    assert total_ids % (8 * NW) == 0
    b_per_w = total_ids // NW
    n_chunks = b_per_w // CHUNK
    mesh = plsc.VectorSubcoreMesh(
        core_axis_name="c", subcore_axis_name="s",
        num_cores=NC, num_subcores=NS)

    @functools.partial(
        pl.kernel,
        mesh=mesh,
        out_type=jax.ShapeDtypeStruct((total_ids, D), jnp.float32),
        scratch_types=[
            pltpu.VMEM((b_per_w,), jnp.int32),
            pltpu.VMEM((CHUNK, D), jnp.float32),
            pltpu.SemaphoreType.DMA,
        ],
    )
    def gather_k(table_hbm, ids_hbm, out_hbm, idx_v, rows_v, sem):
        wid = lax.axis_index("s") * NC + lax.axis_index("c")
        base = wid * b_per_w
        pltpu.sync_copy(ids_hbm.at[pl.ds(base, b_per_w)], idx_v)

        def step(i):
            off = i * CHUNK
            pltpu.async_copy(
                table_hbm.at[idx_v.at[pl.ds(off, CHUNK)]], rows_v, sem
            ).wait()
            pltpu.sync_copy(rows_v, out_hbm.at[pl.ds(base + off, CHUNK)])

        lax.fori_loop(0, n_chunks, lambda i, c: (step(i), c)[1], 0,
                      unroll=False)

    return gather_k


def kernel(input_ids, base_weight, bit_proj_w):
    bsz, seq = input_ids.shape
    table = _build_table(base_weight, bit_proj_w)
    ids = input_ids.reshape(-1).astype(jnp.int32)
    out = _make_gather(bsz * seq)(table, ids)
    return out.reshape(bsz, seq, D)


# speedup vs baseline: 1.5443x; 1.0554x over previous
import functools

import jax
import jax.numpy as jnp
from jax import lax
from jax.experimental import pallas as pl
from jax.experimental.pallas import tpu as pltpu
from jax.experimental.pallas import tpu_sc as plsc

D = 1024
V = 256
NC, NS = 2, 16
NW = NC * NS
CHUNK = 32


def _table_body(base_ref, proj_ref, w_ref):
    r = lax.broadcasted_iota(jnp.int32, (V, 8), 0)
    j = lax.broadcasted_iota(jnp.int32, (V, 8), 1)
    bits = ((r >> (7 - j)) & 1).astype(jnp.float32)
    w_ref[...] = base_ref[...] + lax.dot_general(
        bits, proj_ref[...], (((1,), (1,)), ((), ())),
        preferred_element_type=jnp.float32)


def _build_table(base_weight, bit_proj_w):
    return pl.pallas_call(
        _table_body,
        out_shape=jax.ShapeDtypeStruct((V, D), jnp.float32),
    )(base_weight, bit_proj_w)


def _make_gather(total_ids):
    assert total_ids % (8 * NW) == 0
    b_per_w = total_ids // NW
    n_chunks = b_per_w // CHUNK
    mesh = plsc.VectorSubcoreMesh(
        core_axis_name="c", subcore_axis_name="s",
        num_cores=NC, num_subcores=NS)

    assert n_chunks >= 4 and n_chunks % 2 == 0

    @functools.partial(
        pl.kernel,
        mesh=mesh,
        out_type=jax.ShapeDtypeStruct((total_ids, D), jnp.float32),
        scratch_types=[
            pltpu.VMEM((b_per_w,), jnp.int32),
            pltpu.VMEM((CHUNK, D), jnp.float32),
            pltpu.VMEM((CHUNK, D), jnp.float32),
            pltpu.SemaphoreType.DMA,
            pltpu.SemaphoreType.DMA,
        ],
    )
    def gather_k(table_hbm, ids_hbm, out_hbm, idx_v, buf0, buf1, gsem, ssem):
        wid = lax.axis_index("s") * NC + lax.axis_index("c")
        base = wid * b_per_w
        pltpu.sync_copy(ids_hbm.at[pl.ds(base, b_per_w)], idx_v)

        def start_g(c, buf):
            pltpu.async_copy(
                table_hbm.at[idx_v.at[pl.ds(c * CHUNK, CHUNK)]], buf, gsem)

        def start_s(c, buf):
            pltpu.async_copy(
                buf, out_hbm.at[pl.ds(base + c * CHUNK, CHUNK)], ssem)

        def wait_g():
            pltpu.make_async_copy(
                table_hbm.at[pl.ds(0, CHUNK)], buf0, gsem).wait()

        def wait_s():
            pltpu.make_async_copy(
                buf0, out_hbm.at[pl.ds(base, CHUNK)], ssem).wait()

        start_g(0, buf0)
        wait_g()
        start_g(1, buf1)
        start_s(0, buf0)

        def mid(j, carry):
            c1 = 2 * j + 1
            wait_g()
            wait_s()
            start_g(c1 + 1, buf0)
            start_s(c1, buf1)
            c2 = 2 * j + 2
            wait_g()
            wait_s()
            start_g(c2 + 1, buf1)
            start_s(c2, buf0)
            return carry

        lax.fori_loop(0, (n_chunks - 2) // 2, mid, 0, unroll=False)

        wait_g()
        wait_s()
        start_s(n_chunks - 1, buf1)
        wait_s()

    return gather_k


def kernel(input_ids, base_weight, bit_proj_w):
    bsz, seq = input_ids.shape
    table = _build_table(base_weight, bit_proj_w)
    ids = input_ids.reshape(-1).astype(jnp.int32)
    out = _make_gather(bsz * seq)(table, ids)
    return out.reshape(bsz, seq, D)


# speedup vs baseline: 1.5616x; 1.0112x over previous
import functools

import jax
import jax.numpy as jnp
from jax import lax
from jax.experimental import pallas as pl
from jax.experimental.pallas import tpu as pltpu
from jax.experimental.pallas import tpu_sc as plsc

D = 1024
V = 256
NC, NS = 2, 16
NW = NC * NS
CHUNK = 16
NBUF = 4
CHUNK_BYTES = CHUNK * D * 4


def _table_body(base_ref, proj_ref, w_ref):
    r = lax.broadcasted_iota(jnp.int32, (V, 8), 0)
    j = lax.broadcasted_iota(jnp.int32, (V, 8), 1)
    bits = ((r >> (7 - j)) & 1).astype(jnp.float32)
    w_ref[...] = base_ref[...] + lax.dot_general(
        bits, proj_ref[...], (((1,), (1,)), ((), ())),
        preferred_element_type=jnp.float32)


def _build_table(base_weight, bit_proj_w):
    return pl.pallas_call(
        _table_body,
        out_shape=jax.ShapeDtypeStruct((V, D), jnp.float32),
    )(base_weight, bit_proj_w)


def _make_gather(total_ids):
    assert total_ids % (8 * NW) == 0
    b_per_w = total_ids // NW
    n_chunks = b_per_w // CHUNK
    assert n_chunks % NBUF == 0 and n_chunks >= 2 * NBUF
    mesh = plsc.VectorSubcoreMesh(
        core_axis_name="c", subcore_axis_name="s",
        num_cores=NC, num_subcores=NS)

    @functools.partial(
        pl.kernel,
        mesh=mesh,
        out_type=jax.ShapeDtypeStruct((total_ids, D), jnp.float32),
        scratch_types=[
            pltpu.VMEM((b_per_w,), jnp.int32),
            pltpu.VMEM((CHUNK, D), jnp.float32),
            pltpu.VMEM((CHUNK, D), jnp.float32),
            pltpu.VMEM((CHUNK, D), jnp.float32),
            pltpu.VMEM((CHUNK, D), jnp.float32),
            pltpu.SemaphoreType.DMA,
            pltpu.SemaphoreType.DMA,
        ],
    )
    def gather_k(table_hbm, ids_hbm, out_hbm, idx_v, b0, b1, b2, b3,
                 gsem, ssem):
        bufs = (b0, b1, b2, b3)
        wid = lax.axis_index("s") * NC + lax.axis_index("c")
        base = wid * b_per_w
        pltpu.sync_copy(ids_hbm.at[pl.ds(base, b_per_w)], idx_v)

        def start_g(c, buf):
            pltpu.async_copy(
                table_hbm.at[idx_v.at[pl.ds(c * CHUNK, CHUNK)]], buf, gsem)

        def start_s(c, buf):
            pltpu.async_copy(
                buf, out_hbm.at[pl.ds(base + c * CHUNK, CHUNK)], ssem)

        def wait_g():
            pltpu.make_async_copy(
                table_hbm.at[pl.ds(0, CHUNK)], b0, gsem).wait()

        def wait_s():
            pltpu.make_async_copy(
                b0, out_hbm.at[pl.ds(base, CHUNK)], ssem).wait()

        start_g(0, b0)
        start_g(1, b1)
        wait_g()
        start_s(0, b0)
        start_g(2, b2)
        wait_g()
        start_s(1, b1)
        start_g(3, b3)

        def body(j, carry):
            for b in range(NBUF):
                c = 2 + NBUF * j + b
                wait_g()
                start_s(c, bufs[(b + 2) % NBUF])
                wait_s()
                start_g(c + 2, bufs[b])
            return carry

        lax.fori_loop(0, (n_chunks - 4) // NBUF, body, 0, unroll=False)

        wait_g()
        start_s(n_chunks - 2, bufs[(n_chunks - 2) % NBUF])
        wait_s()
        wait_g()
        start_s(n_chunks - 1, bufs[(n_chunks - 1) % NBUF])
        wait_s()
        wait_s()
        wait_s()

    return gather_k


def kernel(input_ids, base_weight, bit_proj_w):
    bsz, seq = input_ids.shape
    table = _build_table(base_weight, bit_proj_w)
    ids = input_ids.reshape(-1).astype(jnp.int32)
    out = _make_gather(bsz * seq)(table, ids)
    return out.reshape(bsz, seq, D)
